# Initial kernel scaffold; baseline (speedup 1.0000x reference)
import functools

import jax
import jax.numpy as jnp
from jax import lax
from jax.experimental import pallas as pl
from jax.experimental.pallas import tpu as pltpu
from jax.experimental.pallas import tpu_sc as plsc

N = 10000
E = 320000
D = 128

NC = 2
NS = 16
NW = NC * NS
EPW = E // NW
K = 80
NCHUNK = EPW // K
RPS = N // NS
ZR = 125


def _spmm_body(na, x_hbm, *rest):
    edge_refs = rest[: 3 * na]
    out_hbm = rest[3 * na]
    acc, zbuf, colbuf, rowbuf, valbuf, gath, sem = rest[3 * na + 1:]

    c = lax.axis_index("c")
    s = lax.axis_index("s")
    wid = c * NS + s

    def zrow(k, carry):
        for d in range(D // 16):
            zbuf[k, pl.ds(d * 16, 16)] = jnp.zeros((16,), jnp.float32)
        return carry

    lax.fori_loop(0, ZR, zrow, 0)

    for i in range(na):
        rows_hbm = edge_refs[3 * i]
        cols_hbm = edge_refs[3 * i + 1]
        vals_hbm = edge_refs[3 * i + 2]

        for j in range(RPS // ZR):
            pltpu.sync_copy(zbuf, acc.at[pl.ds((s * (RPS // ZR) + j) * ZR, ZR)])
        plsc.subcore_barrier()

        ebase = wid * EPW

        def chunk(t, carry):
            off = pl.multiple_of(ebase + t * K, 8)
            pltpu.sync_copy(cols_hbm.at[pl.ds(off, K)], colbuf)
            pltpu.sync_copy(rows_hbm.at[pl.ds(off, K)], rowbuf)
            pltpu.sync_copy(vals_hbm.at[pl.ds(off, K)], valbuf)
            pltpu.async_copy(x_hbm.at[colbuf], gath, sem).wait()

            def scale(k, c2):
                bc = jnp.full((16,), valbuf[k], jnp.float32)
                for d in range(D // 16):
                    sl = pl.ds(d * 16, 16)
                    gath[k, sl] = gath[k, sl] * bc
                return c2

            lax.fori_loop(0, K, scale, 0)
            pltpu.sync_copy(gath, acc.at[rowbuf], add=True)
            return carry

        lax.fori_loop(0, NCHUNK, chunk, 0)
        plsc.subcore_barrier()

        for j in range(RPS // ZR):
            r0 = (s * (RPS // ZR) + j) * ZR
            pltpu.sync_copy(acc.at[pl.ds(r0, ZR)], out_hbm.at[c, i, pl.ds(r0, ZR)])
        plsc.subcore_barrier()


def _make_spmm(na):
    mesh = plsc.VectorSubcoreMesh(core_axis_name="c", subcore_axis_name="s")
    return pl.kernel(
        functools.partial(_spmm_body, na),
        out_type=jax.ShapeDtypeStruct((NC, na, N, D), jnp.float32),
        mesh=mesh,
        scratch_types=[
            pltpu.VMEM_SHARED((N, D), jnp.float32),
            pltpu.VMEM((ZR, D), jnp.float32),
            pltpu.VMEM((K,), jnp.int32),
            pltpu.VMEM((K,), jnp.int32),
            pltpu.VMEM((K,), jnp.float32),
            pltpu.VMEM((K, D), jnp.float32),
            pltpu.SemaphoreType.DMA,
        ],
        name=f"sc_spmm_{na}",
    )


_spmm4 = _make_spmm(4)
_spmm2 = _make_spmm(2)

BN = 1000


def _mm_body(x_ref, w_ref, b_ref, o_ref):
    o_ref[...] = (
        jnp.dot(x_ref[...], w_ref[...], preferred_element_type=jnp.float32)
        + b_ref[...]
    )


def _matmul(x, w, b):
    return pl.pallas_call(
        _mm_body,
        grid=(N // BN,),
        in_specs=[
            pl.BlockSpec((BN, D), lambda n: (n, 0)),
            pl.BlockSpec((D, D), lambda n: (0, 0)),
            pl.BlockSpec((1, D), lambda n: (0, 0)),
        ],
        out_specs=pl.BlockSpec((BN, D), lambda n: (n, 0)),
        out_shape=jax.ShapeDtypeStruct((N, D), jnp.float32),
    )(x, w, b.reshape(1, D))


def _combo1_body(w_ref, y_ref, o_ref):
    acc = jnp.zeros((BN, D), jnp.float32)
    for i in range(3):
        acc += w_ref[0, i] * (y_ref[0, i] + y_ref[1, i])
    o_ref[...] = acc * (1.0 / 3.0)


def _combo1(w, yp):
    return pl.pallas_call(
        _combo1_body,
        grid=(N // BN,),
        in_specs=[
            pl.BlockSpec(memory_space=pltpu.SMEM),
            pl.BlockSpec((NC, 4, BN, D), lambda n: (0, 0, n, 0)),
        ],
        out_specs=pl.BlockSpec((BN, D), lambda n: (n, 0)),
        out_shape=jax.ShapeDtypeStruct((N, D), jnp.float32),
    )(w.reshape(1, 3), yp)


def _combo2_body(ws_ref, wr_ref, z_ref, y_ref, o_ref):
    acc = jnp.zeros((BN, D), jnp.float32)
    for i in range(3):
        acc += (ws_ref[0, i] / 3.0) * (z_ref[0, i] + z_ref[1, i])
    for i in range(4):
        acc += (wr_ref[0, i] / 4.0) * (y_ref[0, i] + y_ref[1, i])
    o_ref[...] = acc


def _combo2(w_seq_1, w_res_0, zp, yp):
    return pl.pallas_call(
        _combo2_body,
        grid=(N // BN,),
        in_specs=[
            pl.BlockSpec(memory_space=pltpu.SMEM),
            pl.BlockSpec(memory_space=pltpu.SMEM),
            pl.BlockSpec((NC, 4, BN, D), lambda n: (0, 0, n, 0)),
            pl.BlockSpec((NC, 4, BN, D), lambda n: (0, 0, n, 0)),
        ],
        out_specs=pl.BlockSpec((BN, D), lambda n: (n, 0)),
        out_shape=jax.ShapeDtypeStruct((N, D), jnp.float32),
    )(w_seq_1.reshape(1, 3), w_res_0.reshape(1, 4), zp, yp)


def _final_body(wl_ref, w0_ref, w1_ref, u_ref, y_ref, z_ref, o_ref):
    acc = jnp.zeros((BN, D), jnp.float32)
    for i in range(2):
        acc += (wl_ref[0, i] / 2.0) * (u_ref[0, i] + u_ref[1, i])
    for j, i in enumerate((0, 1, 3)):
        acc += (w0_ref[0, j] / 3.0) * (y_ref[0, i] + y_ref[1, i])
        acc += (w1_ref[0, j] / 3.0) * (z_ref[0, i] + z_ref[1, i])
    mu = jnp.mean(acc, axis=-1, keepdims=True)
    ctr = acc - mu
    var = jnp.mean(ctr * ctr, axis=-1, keepdims=True)
    nrm = ctr * lax.rsqrt(var + 1e-5)
    o_ref[...] = 0.5 * nrm * (1.0 + lax.erf(nrm * (2.0 ** -0.5)))


def _final(w_seq_last, w_res_last_0, w_res_last_1, up, yp, zp):
    return pl.pallas_call(
        _final_body,
        grid=(N // BN,),
        in_specs=[
            pl.BlockSpec(memory_space=pltpu.SMEM),
            pl.BlockSpec(memory_space=pltpu.SMEM),
            pl.BlockSpec(memory_space=pltpu.SMEM),
            pl.BlockSpec((NC, 2, BN, D), lambda n: (0, 0, n, 0)),
            pl.BlockSpec((NC, 4, BN, D), lambda n: (0, 0, n, 0)),
            pl.BlockSpec((NC, 4, BN, D), lambda n: (0, 0, n, 0)),
        ],
        out_specs=pl.BlockSpec((BN, D), lambda n: (n, 0)),
        out_shape=jax.ShapeDtypeStruct((N, D), jnp.float32),
    )(
        w_seq_last.reshape(1, 2),
        w_res_last_0.reshape(1, 3),
        w_res_last_1.reshape(1, 3),
        up, yp, zp,
    )


@jax.jit
def kernel(x, affine_w, affine_b,
           adj0_rows, adj0_cols, adj0_vals,
           adj1_rows, adj1_cols, adj1_vals,
           adj2_rows, adj2_cols, adj2_vals,
           adj3_rows, adj3_cols, adj3_vals,
           w_seq_0, w_seq_1, w_seq_last, w_res_0, w_res_last_0, w_res_last_1):
    h = _matmul(x, affine_w, affine_b)
    yp = _spmm4(h,
                adj0_rows, adj0_cols, adj0_vals,
                adj1_rows, adj1_cols, adj1_vals,
                adj2_rows, adj2_cols, adj2_vals,
                adj3_rows, adj3_cols, adj3_vals)
    s1 = _combo1(w_seq_0, yp)
    zp = _spmm4(s1,
                adj0_rows, adj0_cols, adj0_vals,
                adj1_rows, adj1_cols, adj1_vals,
                adj2_rows, adj2_cols, adj2_vals,
                adj3_rows, adj3_cols, adj3_vals)
    s2 = _combo2(w_seq_1, w_res_0, zp, yp)
    up = _spmm2(s2,
                adj0_rows, adj0_cols, adj0_vals,
                adj1_rows, adj1_cols, adj1_vals)
    return _final(w_seq_last, w_res_last_0, w_res_last_1, up, yp, zp)


# speedup vs baseline: 2.6093x; 2.6093x over previous
import functools

import jax
import jax.numpy as jnp
from jax import lax
from jax.experimental import pallas as pl
from jax.experimental.pallas import tpu as pltpu
from jax.experimental.pallas import tpu_sc as plsc

N = 10000
E = 320000
D = 128

NC = 2
NS = 16
NW = NC * NS
EPW = E // NW
K = 80
NCHUNK = EPW // K
NP = 10240
RPS = NP // NS
ZR = 128


def _spmm_body(na, x_hbm, *rest):
    edge_refs = rest[: 3 * na]
    out_hbm = rest[3 * na]
    acc, zbuf, colbuf, rowbuf, valbuf, gath, sem = rest[3 * na + 1:]

    c = lax.axis_index("c")
    s = lax.axis_index("s")
    wid = c * NS + s

    def zrow(k, carry):
        for d in range(D // 16):
            zbuf[k, pl.ds(d * 16, 16)] = jnp.zeros((16,), jnp.float32)
        return carry

    lax.fori_loop(0, ZR, zrow, 0)

    for i in range(na):
        rows_hbm = edge_refs[3 * i]
        cols_hbm = edge_refs[3 * i + 1]
        vals_hbm = edge_refs[3 * i + 2]

        for j in range(RPS // ZR):
            pltpu.sync_copy(zbuf, acc.at[pl.ds((s * (RPS // ZR) + j) * ZR, ZR)])
        plsc.subcore_barrier()

        ebase = wid * EPW

        def chunk(t, carry):
            off = pl.multiple_of(ebase + t * K, 8)
            pltpu.sync_copy(cols_hbm.at[pl.ds(off, K)], colbuf)
            pltpu.sync_copy(rows_hbm.at[pl.ds(off, K)], rowbuf)
            pltpu.sync_copy(vals_hbm.at[pl.ds(off, K)], valbuf)
            pltpu.async_copy(x_hbm.at[colbuf], gath, sem).wait()

            def scale(q, c2):
                vals16 = valbuf[pl.ds(q * 16, 16)]
                for j in range(16):
                    k = q * 16 + j
                    bc = jnp.full((16,), vals16[j], jnp.float32)
                    for d in range(D // 16):
                        sl = pl.ds(d * 16, 16)
                        gath[k, sl] = gath[k, sl] * bc
                return c2

            lax.fori_loop(0, K // 16, scale, 0)
            pltpu.sync_copy(gath, acc.at[rowbuf], add=True)
            return carry

        lax.fori_loop(0, NCHUNK, chunk, 0)
        plsc.subcore_barrier()

        for j in range(RPS // ZR):
            r0 = (s * (RPS // ZR) + j) * ZR
            pltpu.sync_copy(acc.at[pl.ds(r0, ZR)], out_hbm.at[c, i, pl.ds(r0, ZR)])
        plsc.subcore_barrier()


def _make_spmm(na):
    mesh = plsc.VectorSubcoreMesh(core_axis_name="c", subcore_axis_name="s")
    return pl.kernel(
        functools.partial(_spmm_body, na),
        out_type=jax.ShapeDtypeStruct((NC, na, NP, D), jnp.float32),
        mesh=mesh,
        scratch_types=[
            pltpu.VMEM_SHARED((NP, D), jnp.float32),
            pltpu.VMEM((ZR, D), jnp.float32),
            pltpu.VMEM((K,), jnp.int32),
            pltpu.VMEM((K,), jnp.int32),
            pltpu.VMEM((K,), jnp.float32),
            pltpu.VMEM((K, D), jnp.float32),
            pltpu.SemaphoreType.DMA,
        ],
        name=f"sc_spmm_{na}",
    )


_spmm4 = _make_spmm(4)
_spmm2 = _make_spmm(2)

BN = 1000
BNP = 1024


def _mm_body(x_ref, w_ref, b_ref, o_ref):
    o_ref[...] = (
        jnp.dot(x_ref[...], w_ref[...], preferred_element_type=jnp.float32)
        + b_ref[...]
    )


def _matmul(x, w, b):
    return pl.pallas_call(
        _mm_body,
        grid=(N // BN,),
        in_specs=[
            pl.BlockSpec((BN, D), lambda n: (n, 0)),
            pl.BlockSpec((D, D), lambda n: (0, 0)),
            pl.BlockSpec((1, D), lambda n: (0, 0)),
        ],
        out_specs=pl.BlockSpec((BN, D), lambda n: (n, 0)),
        out_shape=jax.ShapeDtypeStruct((N, D), jnp.float32),
    )(x, w, b.reshape(1, D))


def _combo1_body(w_ref, y_ref, o_ref):
    acc = jnp.zeros((BNP, D), jnp.float32)
    for i in range(3):
        acc += w_ref[0, i] * (y_ref[0, i] + y_ref[1, i])
    o_ref[...] = acc * (1.0 / 3.0)


def _combo1(w, yp):
    return pl.pallas_call(
        _combo1_body,
        grid=(NP // BNP,),
        in_specs=[
            pl.BlockSpec(memory_space=pltpu.SMEM),
            pl.BlockSpec((NC, 4, BNP, D), lambda n: (0, 0, n, 0)),
        ],
        out_specs=pl.BlockSpec((BNP, D), lambda n: (n, 0)),
        out_shape=jax.ShapeDtypeStruct((NP, D), jnp.float32),
    )(w.reshape(1, 3), yp)


def _combo2_body(ws_ref, wr_ref, z_ref, y_ref, o_ref):
    acc = jnp.zeros((BNP, D), jnp.float32)
    for i in range(3):
        acc += (ws_ref[0, i] / 3.0) * (z_ref[0, i] + z_ref[1, i])
    for i in range(4):
        acc += (wr_ref[0, i] / 4.0) * (y_ref[0, i] + y_ref[1, i])
    o_ref[...] = acc


def _combo2(w_seq_1, w_res_0, zp, yp):
    return pl.pallas_call(
        _combo2_body,
        grid=(NP // BNP,),
        in_specs=[
            pl.BlockSpec(memory_space=pltpu.SMEM),
            pl.BlockSpec(memory_space=pltpu.SMEM),
            pl.BlockSpec((NC, 4, BNP, D), lambda n: (0, 0, n, 0)),
            pl.BlockSpec((NC, 4, BNP, D), lambda n: (0, 0, n, 0)),
        ],
        out_specs=pl.BlockSpec((BNP, D), lambda n: (n, 0)),
        out_shape=jax.ShapeDtypeStruct((NP, D), jnp.float32),
    )(w_seq_1.reshape(1, 3), w_res_0.reshape(1, 4), zp, yp)


def _final_body(wl_ref, w0_ref, w1_ref, u_ref, y_ref, z_ref, o_ref):
    acc = jnp.zeros((BNP, D), jnp.float32)
    for i in range(2):
        acc += (wl_ref[0, i] / 2.0) * (u_ref[0, i] + u_ref[1, i])
    for j, i in enumerate((0, 1, 3)):
        acc += (w0_ref[0, j] / 3.0) * (y_ref[0, i] + y_ref[1, i])
        acc += (w1_ref[0, j] / 3.0) * (z_ref[0, i] + z_ref[1, i])
    mu = jnp.mean(acc, axis=-1, keepdims=True)
    ctr = acc - mu
    var = jnp.mean(ctr * ctr, axis=-1, keepdims=True)
    nrm = ctr * lax.rsqrt(var + 1e-5)
    o_ref[...] = 0.5 * nrm * (1.0 + lax.erf(nrm * (2.0 ** -0.5)))


def _final(w_seq_last, w_res_last_0, w_res_last_1, up, yp, zp):
    return pl.pallas_call(
        _final_body,
        grid=(NP // BNP,),
        in_specs=[
            pl.BlockSpec(memory_space=pltpu.SMEM),
            pl.BlockSpec(memory_space=pltpu.SMEM),
            pl.BlockSpec(memory_space=pltpu.SMEM),
            pl.BlockSpec((NC, 2, BNP, D), lambda n: (0, 0, n, 0)),
            pl.BlockSpec((NC, 4, BNP, D), lambda n: (0, 0, n, 0)),
            pl.BlockSpec((NC, 4, BNP, D), lambda n: (0, 0, n, 0)),
        ],
        out_specs=pl.BlockSpec((BNP, D), lambda n: (n, 0)),
        out_shape=jax.ShapeDtypeStruct((NP, D), jnp.float32),
    )(
        w_seq_last.reshape(1, 2),
        w_res_last_0.reshape(1, 3),
        w_res_last_1.reshape(1, 3),
        up, yp, zp,
    )


@jax.jit
def kernel(x, affine_w, affine_b,
           adj0_rows, adj0_cols, adj0_vals,
           adj1_rows, adj1_cols, adj1_vals,
           adj2_rows, adj2_cols, adj2_vals,
           adj3_rows, adj3_cols, adj3_vals,
           w_seq_0, w_seq_1, w_seq_last, w_res_0, w_res_last_0, w_res_last_1):
    h = _matmul(x, affine_w, affine_b)
    yp = _spmm4(h,
                adj0_rows, adj0_cols, adj0_vals,
                adj1_rows, adj1_cols, adj1_vals,
                adj2_rows, adj2_cols, adj2_vals,
                adj3_rows, adj3_cols, adj3_vals)
    s1 = _combo1(w_seq_0, yp)
    zp = _spmm4(s1,
                adj0_rows, adj0_cols, adj0_vals,
                adj1_rows, adj1_cols, adj1_vals,
                adj2_rows, adj2_cols, adj2_vals,
                adj3_rows, adj3_cols, adj3_vals)
    s2 = _combo2(w_seq_1, w_res_0, zp, yp)
    up = _spmm2(s2,
                adj0_rows, adj0_cols, adj0_vals,
                adj1_rows, adj1_cols, adj1_vals)
    out = _final(w_seq_last, w_res_last_0, w_res_last_1, up, yp, zp)
    return out[:N]


# speedup vs baseline: 7.9862x; 3.0606x over previous
import functools

import jax
import jax.numpy as jnp
from jax import lax
from jax.experimental import pallas as pl
from jax.experimental.pallas import tpu as pltpu
from jax.experimental.pallas import tpu_sc as plsc

N = 10000
E = 320000
D = 128

NC = 2
NS = 16
NW = NC * NS
EPW = E // NW
K = 80
NCHUNK = EPW // K
NP = 10240
RPS = NP // NS
ZR = 128


RING = 4


def _spmm_body(na, x_hbm, z_hbm, *rest):
    edge_refs = rest[: 3 * na]
    out_hbm = rest[3 * na]
    scr = rest[3 * na + 1:]
    acc = scr[0]
    rowb = scr[1:1 + RING]
    colb = scr[1 + RING:1 + 2 * RING]
    valb = scr[1 + 2 * RING:1 + 3 * RING]
    gath = scr[1 + 3 * RING:1 + 4 * RING]
    seme = scr[1 + 4 * RING:1 + 5 * RING]
    semg = scr[1 + 5 * RING:1 + 6 * RING]
    sems = scr[1 + 6 * RING:1 + 7 * RING]
    semz = scr[1 + 7 * RING]

    c = lax.axis_index("c")
    s = lax.axis_index("s")
    wid = c * NS + s
    r0 = s * RPS

    def start_idx(er, ec, ev, t, b):
        pltpu.async_copy(er.at[wid, t], rowb[b], seme[b])
        pltpu.async_copy(ec.at[wid, t], colb[b], seme[b])
        pltpu.async_copy(ev.at[wid, t], valb[b], seme[b])

    def wait_idx(er, ec, ev, t, b):
        pltpu.make_async_copy(er.at[wid, t], rowb[b], seme[b]).wait()
        pltpu.make_async_copy(ec.at[wid, t], colb[b], seme[b]).wait()
        pltpu.make_async_copy(ev.at[wid, t], valb[b], seme[b]).wait()

    def start_gather(b):
        pltpu.async_copy(x_hbm.at[colb[b]], gath[b], semg[b])

    def wait_gather(b):
        pltpu.make_async_copy(x_hbm.at[colb[b]], gath[b], semg[b]).wait()

    def start_scatter(b):
        pltpu.async_copy(gath[b], acc.at[rowb[b]], sems[b], add=True)

    def wait_scatter(b):
        pltpu.make_async_copy(gath[b], acc.at[rowb[b]], sems[b]).wait()

    def scale(b):
        def scale16(q, c2):
            v16 = valb[b][pl.ds(q * 16, 16)]
            for j in range(16):
                k = q * 16 + j
                bc = jnp.full((16,), v16[j], jnp.float32)
                for d in range(D // 16):
                    sl = pl.ds(d * 16, 16)
                    gath[b][k, sl] = gath[b][k, sl] * bc
            return c2

        lax.fori_loop(0, K // 16, scale16, 0)

    for i in range(na):
        er = edge_refs[3 * i]
        ec = edge_refs[3 * i + 1]
        ev = edge_refs[3 * i + 2]

        pltpu.async_copy(z_hbm.at[pl.ds(r0, RPS)], acc.at[pl.ds(r0, RPS)], semz)
        start_idx(er, ec, ev, 0, 0)
        start_idx(er, ec, ev, 1, 1)
        wait_idx(er, ec, ev, 0, 0)
        start_gather(0)
        pltpu.make_async_copy(
            z_hbm.at[pl.ds(r0, RPS)], acc.at[pl.ds(r0, RPS)], semz).wait()
        plsc.subcore_barrier()

        def step(t4, carry):
            for u in range(RING):
                t = t4 * RING + u
                b = u

                @pl.when(jnp.logical_and(t >= 2, t < NCHUNK + 2))
                def _():
                    wait_scatter((u + 2) % RING)

                @pl.when(t + 2 < NCHUNK)
                def _():
                    start_idx(er, ec, ev, t + 2, (u + 2) % RING)

                @pl.when(t + 1 < NCHUNK)
                def _():
                    wait_idx(er, ec, ev, t + 1, (u + 1) % RING)
                    start_gather((u + 1) % RING)

                @pl.when(t < NCHUNK)
                def _():
                    wait_gather(b)
                    scale(b)
                    start_scatter(b)
            return carry

        nsteps = (NCHUNK + 2 + RING - 1) // RING
        lax.fori_loop(0, nsteps, step, 0)
        plsc.subcore_barrier()

        pltpu.sync_copy(acc.at[pl.ds(r0, RPS)], out_hbm.at[c, i, pl.ds(r0, RPS)])
        plsc.subcore_barrier()


def _make_spmm(na):
    mesh = plsc.VectorSubcoreMesh(core_axis_name="c", subcore_axis_name="s")
    scratch = [pltpu.VMEM_SHARED((NP, D), jnp.float32)]
    scratch += [pltpu.VMEM((K,), jnp.int32) for _ in range(RING)]
    scratch += [pltpu.VMEM((K,), jnp.int32) for _ in range(RING)]
    scratch += [pltpu.VMEM((K,), jnp.float32) for _ in range(RING)]
    scratch += [pltpu.VMEM((K, D), jnp.float32) for _ in range(RING)]
    scratch += [pltpu.SemaphoreType.DMA for _ in range(3 * RING + 1)]
    return pl.kernel(
        functools.partial(_spmm_body, na),
        out_type=jax.ShapeDtypeStruct((NC, na, NP, D), jnp.float32),
        mesh=mesh,
        scratch_types=scratch,
        name=f"sc_spmm_{na}",
    )


_spmm4 = _make_spmm(4)
_spmm2 = _make_spmm(2)

BN = 1000
BNP = 1024


def _mm_body(x_ref, w_ref, b_ref, o_ref):
    o_ref[...] = (
        jnp.dot(x_ref[...], w_ref[...], preferred_element_type=jnp.float32)
        + b_ref[...]
    )


def _matmul(x, w, b):
    return pl.pallas_call(
        _mm_body,
        grid=(N // BN,),
        in_specs=[
            pl.BlockSpec((BN, D), lambda n: (n, 0)),
            pl.BlockSpec((D, D), lambda n: (0, 0)),
            pl.BlockSpec((1, D), lambda n: (0, 0)),
        ],
        out_specs=pl.BlockSpec((BN, D), lambda n: (n, 0)),
        out_shape=jax.ShapeDtypeStruct((N, D), jnp.float32),
    )(x, w, b.reshape(1, D))


def _combo1_body(w_ref, y_ref, o_ref):
    acc = jnp.zeros((BNP, D), jnp.float32)
    for i in range(3):
        acc += w_ref[0, i] * (y_ref[0, i] + y_ref[1, i])
    o_ref[...] = acc * (1.0 / 3.0)


def _combo1(w, yp):
    return pl.pallas_call(
        _combo1_body,
        grid=(NP // BNP,),
        in_specs=[
            pl.BlockSpec(memory_space=pltpu.SMEM),
            pl.BlockSpec((NC, 4, BNP, D), lambda n: (0, 0, n, 0)),
        ],
        out_specs=pl.BlockSpec((BNP, D), lambda n: (n, 0)),
        out_shape=jax.ShapeDtypeStruct((NP, D), jnp.float32),
    )(w.reshape(1, 3), yp)


def _combo2_body(ws_ref, wr_ref, z_ref, y_ref, o_ref):
    acc = jnp.zeros((BNP, D), jnp.float32)
    for i in range(3):
        acc += (ws_ref[0, i] / 3.0) * (z_ref[0, i] + z_ref[1, i])
    for i in range(4):
        acc += (wr_ref[0, i] / 4.0) * (y_ref[0, i] + y_ref[1, i])
    o_ref[...] = acc


def _combo2(w_seq_1, w_res_0, zp, yp):
    return pl.pallas_call(
        _combo2_body,
        grid=(NP // BNP,),
        in_specs=[
            pl.BlockSpec(memory_space=pltpu.SMEM),
            pl.BlockSpec(memory_space=pltpu.SMEM),
            pl.BlockSpec((NC, 4, BNP, D), lambda n: (0, 0, n, 0)),
            pl.BlockSpec((NC, 4, BNP, D), lambda n: (0, 0, n, 0)),
        ],
        out_specs=pl.BlockSpec((BNP, D), lambda n: (n, 0)),
        out_shape=jax.ShapeDtypeStruct((NP, D), jnp.float32),
    )(w_seq_1.reshape(1, 3), w_res_0.reshape(1, 4), zp, yp)


def _final_body(wl_ref, w0_ref, w1_ref, u_ref, y_ref, z_ref, o_ref):
    acc = jnp.zeros((BNP, D), jnp.float32)
    for i in range(2):
        acc += (wl_ref[0, i] / 2.0) * (u_ref[0, i] + u_ref[1, i])
    for j, i in enumerate((0, 1, 3)):
        acc += (w0_ref[0, j] / 3.0) * (y_ref[0, i] + y_ref[1, i])
        acc += (w1_ref[0, j] / 3.0) * (z_ref[0, i] + z_ref[1, i])
    mu = jnp.mean(acc, axis=-1, keepdims=True)
    ctr = acc - mu
    var = jnp.mean(ctr * ctr, axis=-1, keepdims=True)
    nrm = ctr * lax.rsqrt(var + 1e-5)
    o_ref[...] = 0.5 * nrm * (1.0 + lax.erf(nrm * (2.0 ** -0.5)))


def _final(w_seq_last, w_res_last_0, w_res_last_1, up, yp, zp):
    return pl.pallas_call(
        _final_body,
        grid=(NP // BNP,),
        in_specs=[
            pl.BlockSpec(memory_space=pltpu.SMEM),
            pl.BlockSpec(memory_space=pltpu.SMEM),
            pl.BlockSpec(memory_space=pltpu.SMEM),
            pl.BlockSpec((NC, 2, BNP, D), lambda n: (0, 0, n, 0)),
            pl.BlockSpec((NC, 4, BNP, D), lambda n: (0, 0, n, 0)),
            pl.BlockSpec((NC, 4, BNP, D), lambda n: (0, 0, n, 0)),
        ],
        out_specs=pl.BlockSpec((BNP, D), lambda n: (n, 0)),
        out_shape=jax.ShapeDtypeStruct((NP, D), jnp.float32),
    )(
        w_seq_last.reshape(1, 2),
        w_res_last_0.reshape(1, 3),
        w_res_last_1.reshape(1, 3),
        up, yp, zp,
    )


@jax.jit
def kernel(x, affine_w, affine_b,
           adj0_rows, adj0_cols, adj0_vals,
           adj1_rows, adj1_cols, adj1_vals,
           adj2_rows, adj2_cols, adj2_vals,
           adj3_rows, adj3_cols, adj3_vals,
           w_seq_0, w_seq_1, w_seq_last, w_res_0, w_res_last_0, w_res_last_1):
    eshape = (NW, NCHUNK, K)

    def pack(r, cc, v):
        return (r.reshape(eshape), cc.reshape(eshape), v.reshape(eshape))

    e0 = pack(adj0_rows, adj0_cols, adj0_vals)
    e1 = pack(adj1_rows, adj1_cols, adj1_vals)
    e2 = pack(adj2_rows, adj2_cols, adj2_vals)
    e3 = pack(adj3_rows, adj3_cols, adj3_vals)
    z = jnp.zeros((NP, D), jnp.float32)
    h = _matmul(x, affine_w, affine_b)
    yp = _spmm4(h, z, *e0, *e1, *e2, *e3)
    s1 = _combo1(w_seq_0, yp)
    zp = _spmm4(s1, z, *e0, *e1, *e2, *e3)
    s2 = _combo2(w_seq_1, w_res_0, zp, yp)
    up = _spmm2(s2, z, *e0, *e1)
    out = _final(w_seq_last, w_res_last_0, w_res_last_1, up, yp, zp)
    return out[:N]
